# baseline (speedup 1.0000x reference)
import functools

import jax
import jax.numpy as jnp
from jax import lax
from jax.experimental import pallas as pl
from jax.experimental.pallas import tpu as pltpu
from jax.experimental.pallas import tpu_sc as plsc

N = 10000
E = 320000
D = 128
NC = 2
NS = 16
EPT = E // NS
CHUNK = 40
NCHUNK = EPT // CHUNK
PHASES = 5
PH = NCHUNK // PHASES
NBUF = 4
CH_D = 125
PHASES_D = 4
PH_D = EPT // CH_D // PHASES_D
ROWS_PT = N // NS
N_PAD = 640 * NS

def _deg_body(dsts_hbm, ones1_hbm, ones2_hbm, deg_hbm, idx_v, ones_v, deg_sp):
    g = lax.axis_index("c")
    s = lax.axis_index("s")
    pltpu.sync_copy(ones1_hbm, deg_sp.at[pl.ds(s * 640, 640)])
    pltpu.sync_copy(ones2_hbm, ones_v)
    plsc.subcore_barrier()

    for phase in range(PHASES_D):
        pltpu.sync_copy(dsts_hbm.at[g, s, phase], idx_v)

        def step(j, _):
            pltpu.sync_copy(
                ones_v.at[pl.ds(0, CH_D)],
                deg_sp.at[idx_v.at[j]],
                add=True,
            )
            return _

        lax.fori_loop(0, PH_D, step, None)
    plsc.subcore_barrier()

    @pl.when(s == 0)
    def _():
        pltpu.sync_copy(deg_sp, deg_hbm.at[g])


def _prop_body(hp_hbm, srcs_hbm, dsts_hbm, acc_hbm,
               src_idx, dst_idx, buf0, buf1, buf2, buf3,
               gs0, gs1, gs2, gs3, ss0, ss1, ss2, ss3, acc_sp):
    g = lax.axis_index("c")
    s = lax.axis_index("s")
    hp_g = hp_hbm.at[g]

    @pl.when(s < NS - 1)
    def _():
        pltpu.sync_copy(hp_g.at[pl.ds(s * 632, 632)],
                        acc_sp.at[pl.ds(s * 632, 632)])

    @pl.when(s == NS - 1)
    def _():
        pltpu.sync_copy(hp_g.at[pl.ds(9480, 520)],
                        acc_sp.at[pl.ds(9480, 520)])

    plsc.subcore_barrier()

    bufs = (buf0, buf1, buf2, buf3)
    gsems = (gs0, gs1, gs2, gs3)
    ssems = (ss0, ss1, ss2, ss3)
    NI = PH // NBUF

    for phase in range(PHASES):
        pltpu.sync_copy(srcs_hbm.at[g, s, phase], src_idx)
        pltpu.sync_copy(dsts_hbm.at[g, s, phase], dst_idx)
        for k in range(NBUF - 1):
            pltpu.async_copy(hp_g.at[src_idx.at[k]], bufs[k], gsems[k])

        def step(i, _):
            for k in range(NBUF):
                j = NBUF * i + k
                pltpu.make_async_copy(
                    hp_g.at[src_idx.at[0]], bufs[k], gsems[k]).wait()
                pltpu.async_copy(bufs[k], acc_sp.at[dst_idx.at[j]],
                                 ssems[k], add=True)
                kn = (k + 3) % NBUF
                if k == 0:
                    @pl.when(i == 0)
                    def _():
                        pltpu.async_copy(hp_g.at[src_idx.at[3]],
                                         bufs[3], gsems[3])

                    @pl.when(i > 0)
                    def _():
                        pltpu.make_async_copy(
                            bufs[kn], acc_sp.at[dst_idx.at[0]],
                            ssems[kn]).wait()
                        pltpu.async_copy(hp_g.at[src_idx.at[j + 3]],
                                         bufs[kn], gsems[kn])
                else:
                    @pl.when(i < NI - 1)
                    def _():
                        pltpu.make_async_copy(
                            bufs[kn], acc_sp.at[dst_idx.at[0]],
                            ssems[kn]).wait()
                        pltpu.async_copy(hp_g.at[src_idx.at[j + 3]],
                                         bufs[kn], gsems[kn])
            return _

        lax.fori_loop(0, NI, step, None)
        for k in range(NBUF):
            pltpu.make_async_copy(bufs[k], acc_sp.at[dst_idx.at[0]],
                                  ssems[k]).wait()

    plsc.subcore_barrier()

    @pl.when(s < NS - 1)
    def _():
        pltpu.sync_copy(acc_sp.at[pl.ds(s * 632, 632)],
                        acc_hbm.at[g].at[pl.ds(s * 632, 632)])

    @pl.when(s == NS - 1)
    def _():
        pltpu.sync_copy(acc_sp.at[pl.ds(9480, 520)],
                        acc_hbm.at[g].at[pl.ds(9480, 520)])


@functools.cache
def _sc_kernels():
    mesh = plsc.VectorSubcoreMesh(
        core_axis_name="c", subcore_axis_name="s", num_cores=NC, num_subcores=NS
    )
    deg_k = pl.kernel(
        _deg_body,
        out_type=jax.ShapeDtypeStruct((NC, N_PAD), jnp.float32),
        mesh=mesh,
        scratch_types=[
            pltpu.VMEM((PH_D, CH_D), jnp.int32),
            pltpu.VMEM((128,), jnp.float32),
            pltpu.VMEM_SHARED((N_PAD,), jnp.float32),
        ],
    )
    prop_k = pl.kernel(
        _prop_body,
        out_type=jax.ShapeDtypeStruct((NC, N, D), jnp.float32),
        mesh=mesh,
        scratch_types=(
            [pltpu.VMEM((PH, CHUNK), jnp.int32)] * 2
            + [pltpu.VMEM((CHUNK, D), jnp.float32)] * NBUF
            + [pltpu.SemaphoreType.DMA] * (2 * NBUF)
            + [pltpu.VMEM_SHARED((N, D), jnp.float32)]
        ),
    )
    return deg_k, prop_k


BLK = 1000
GRID = 2 * N // BLK


def _tc1_body(x_ref, w_ref, bl_ref, deg_ref, init_ref, hp_ref, dinv_ref):
    y = jnp.dot(x_ref[...], w_ref[...], preferred_element_type=jnp.float32)
    init_ref[...] = y[:, :D] + bl_ref[...][None, :]
    dinv = lax.rsqrt(deg_ref[...])
    dinv_ref[...] = dinv
    hp_ref[...] = y[:, D:] * dinv


_tc1 = pl.pallas_call(
    _tc1_body,
    grid=(GRID,),
    in_specs=[
        pl.BlockSpec((BLK, D), lambda i: (i, 0)),
        pl.BlockSpec((D, 2 * D), lambda i: (0, 0)),
        pl.BlockSpec((D,), lambda i: (0,)),
        pl.BlockSpec((BLK, D), lambda i: (i, 0)),
    ],
    out_specs=[
        pl.BlockSpec((BLK, D), lambda i: (i, 0)),
        pl.BlockSpec((BLK, D), lambda i: (i, 0)),
        pl.BlockSpec((BLK, D), lambda i: (i, 0)),
    ],
    out_shape=[
        jax.ShapeDtypeStruct((2 * N, D), jnp.float32),
        jax.ShapeDtypeStruct((2 * N, D), jnp.float32),
        jax.ShapeDtypeStruct((2 * N, D), jnp.float32),
    ],
)


def _tc2_body(acc_ref, dinv_ref, init_ref, b_ref, wg_ref, hp_ref):
    dinv = dinv_ref[...]
    p = jax.nn.relu(acc_ref[...] * dinv + b_ref[...][None, :])
    z = 0.5 * p + 0.5 * init_ref[...]
    hp_ref[...] = jnp.dot(z, wg_ref[...], preferred_element_type=jnp.float32) * dinv


_tc2 = pl.pallas_call(
    _tc2_body,
    grid=(GRID,),
    in_specs=[
        pl.BlockSpec((BLK, D), lambda i: (i, 0)),
        pl.BlockSpec((BLK, D), lambda i: (i, 0)),
        pl.BlockSpec((BLK, D), lambda i: (i, 0)),
        pl.BlockSpec((D,), lambda i: (0,)),
        pl.BlockSpec((D, D), lambda i: (0, 0)),
    ],
    out_specs=pl.BlockSpec((BLK, D), lambda i: (i, 0)),
    out_shape=jax.ShapeDtypeStruct((2 * N, D), jnp.float32),
)


def _tc3_body(acc_ref, dinv_ref, init_ref, b_ref, out_ref):
    p = jax.nn.relu(acc_ref[...] * dinv_ref[...] + b_ref[...][None, :])
    y = 0.5 * p + 0.5 * init_ref[...]
    nrm = jnp.sqrt(jnp.sum(y * y, axis=1, keepdims=True))
    out_ref[...] = y / jnp.maximum(nrm, 1e-12)


_tc3 = pl.pallas_call(
    _tc3_body,
    grid=(GRID,),
    in_specs=[
        pl.BlockSpec((BLK, D), lambda i: (i, 0)),
        pl.BlockSpec((BLK, D), lambda i: (i, 0)),
        pl.BlockSpec((BLK, D), lambda i: (i, 0)),
        pl.BlockSpec((D,), lambda i: (0,)),
    ],
    out_specs=pl.BlockSpec((BLK, D), lambda i: (i, 0)),
    out_shape=jax.ShapeDtypeStruct((2 * N, D), jnp.float32),
)


def kernel(x1, edge_index1, x2, edge_index2, W_lin, b_lin, W_in, b_in, W_g, b_g):
    srcs = jnp.stack([edge_index1[0], edge_index2[0]]).reshape(
        NC, NS, PHASES, PH, CHUNK)
    dsts = jnp.stack([edge_index1[1], edge_index2[1]]).reshape(
        NC, NS, PHASES, PH, CHUNK)
    dsts_deg = jnp.stack([edge_index1[1], edge_index2[1]]).reshape(
        NC, NS, PHASES_D, PH_D, CH_D)
    ones1 = jnp.ones((640,), jnp.float32)
    ones2 = jnp.ones((128,), jnp.float32)
    _deg_kernel, _prop_kernel = _sc_kernels()

    deg = _deg_kernel(dsts_deg, ones1, ones2)[:, :N]
    deg_bc = jnp.broadcast_to(deg.reshape(2 * N, 1), (2 * N, D))

    x_all = jnp.concatenate([x1, x2], axis=0)
    w_cat = jnp.concatenate([W_lin, W_in], axis=1)
    init, hp0, dinv = _tc1(x_all, w_cat, b_lin, deg_bc)

    def layer(i, carry):
        hp, _ = carry
        acc = _prop_kernel(hp.reshape(NC, N, D), srcs, dsts).reshape(2 * N, D)
        b = jnp.where(i == 0, b_in, b_g)
        hp_next = _tc2(acc, dinv, init, b, W_g)
        return hp_next, acc

    _, acc3 = lax.fori_loop(0, 3, layer, (hp0, jnp.zeros_like(hp0)))
    out = _tc3(acc3, dinv, init, b_g)

    return out[:N], out[N:]


# speedup vs baseline: 1.0208x; 1.0208x over previous
import functools

import jax
import jax.numpy as jnp
from jax import lax
from jax.experimental import pallas as pl
from jax.experimental.pallas import tpu as pltpu
from jax.experimental.pallas import tpu_sc as plsc

N = 10000
E = 320000
D = 128
NC = 2
NS = 16
EPT = E // NS
CHUNK = 50
NCHUNK = EPT // CHUNK
PHASES = 5
PH = NCHUNK // PHASES
NBUF = 4
CH_D = 125
PHASES_D = 4
PH_D = EPT // CH_D // PHASES_D
ROWS_PT = N // NS
N_PAD = 640 * NS

def _deg_body(dsts_hbm, ones1_hbm, ones2_hbm, deg_hbm, idx_v, ones_v, deg_sp):
    g = lax.axis_index("c")
    s = lax.axis_index("s")
    pltpu.sync_copy(ones1_hbm, deg_sp.at[pl.ds(s * 640, 640)])
    pltpu.sync_copy(ones2_hbm, ones_v)
    plsc.subcore_barrier()

    for phase in range(PHASES_D):
        pltpu.sync_copy(dsts_hbm.at[g, s, phase], idx_v)

        def step(j, _):
            pltpu.sync_copy(
                ones_v.at[pl.ds(0, CH_D)],
                deg_sp.at[idx_v.at[j]],
                add=True,
            )
            return _

        lax.fori_loop(0, PH_D, step, None)
    plsc.subcore_barrier()

    @pl.when(s == 0)
    def _():
        pltpu.sync_copy(deg_sp, deg_hbm.at[g])


def _prop_body(hp_hbm, srcs_hbm, dsts_hbm, acc_hbm,
               src_idx, dst_idx, buf0, buf1, buf2, buf3,
               gs0, gs1, gs2, gs3, ss0, ss1, ss2, ss3, acc_sp):
    g = lax.axis_index("c")
    s = lax.axis_index("s")
    hp_g = hp_hbm.at[g]

    @pl.when(s < NS - 1)
    def _():
        pltpu.sync_copy(hp_g.at[pl.ds(s * 632, 632)],
                        acc_sp.at[pl.ds(s * 632, 632)])

    @pl.when(s == NS - 1)
    def _():
        pltpu.sync_copy(hp_g.at[pl.ds(9480, 520)],
                        acc_sp.at[pl.ds(9480, 520)])

    plsc.subcore_barrier()

    bufs = (buf0, buf1, buf2, buf3)
    gsems = (gs0, gs1, gs2, gs3)
    ssems = (ss0, ss1, ss2, ss3)
    NI = PH // NBUF

    for phase in range(PHASES):
        pltpu.sync_copy(srcs_hbm.at[g, s, phase], src_idx)
        pltpu.sync_copy(dsts_hbm.at[g, s, phase], dst_idx)
        for k in range(NBUF - 1):
            pltpu.async_copy(hp_g.at[src_idx.at[k]], bufs[k], gsems[k])

        def step(i, _):
            for k in range(NBUF):
                j = NBUF * i + k
                pltpu.make_async_copy(
                    hp_g.at[src_idx.at[0]], bufs[k], gsems[k]).wait()
                pltpu.async_copy(bufs[k], acc_sp.at[dst_idx.at[j]],
                                 ssems[k], add=True)
                kn = (k + 3) % NBUF
                if k == 0:
                    @pl.when(i == 0)
                    def _():
                        pltpu.async_copy(hp_g.at[src_idx.at[3]],
                                         bufs[3], gsems[3])

                    @pl.when(i > 0)
                    def _():
                        pltpu.make_async_copy(
                            bufs[kn], acc_sp.at[dst_idx.at[0]],
                            ssems[kn]).wait()
                        pltpu.async_copy(hp_g.at[src_idx.at[j + 3]],
                                         bufs[kn], gsems[kn])
                else:
                    @pl.when(i < NI - 1)
                    def _():
                        pltpu.make_async_copy(
                            bufs[kn], acc_sp.at[dst_idx.at[0]],
                            ssems[kn]).wait()
                        pltpu.async_copy(hp_g.at[src_idx.at[j + 3]],
                                         bufs[kn], gsems[kn])
            return _

        lax.fori_loop(0, NI, step, None)
        for k in range(NBUF):
            pltpu.make_async_copy(bufs[k], acc_sp.at[dst_idx.at[0]],
                                  ssems[k]).wait()

    plsc.subcore_barrier()

    @pl.when(s < NS - 1)
    def _():
        pltpu.sync_copy(acc_sp.at[pl.ds(s * 632, 632)],
                        acc_hbm.at[g].at[pl.ds(s * 632, 632)])

    @pl.when(s == NS - 1)
    def _():
        pltpu.sync_copy(acc_sp.at[pl.ds(9480, 520)],
                        acc_hbm.at[g].at[pl.ds(9480, 520)])


@functools.cache
def _sc_kernels():
    mesh = plsc.VectorSubcoreMesh(
        core_axis_name="c", subcore_axis_name="s", num_cores=NC, num_subcores=NS
    )
    deg_k = pl.kernel(
        _deg_body,
        out_type=jax.ShapeDtypeStruct((NC, N_PAD), jnp.float32),
        mesh=mesh,
        scratch_types=[
            pltpu.VMEM((PH_D, CH_D), jnp.int32),
            pltpu.VMEM((128,), jnp.float32),
            pltpu.VMEM_SHARED((N_PAD,), jnp.float32),
        ],
    )
    prop_k = pl.kernel(
        _prop_body,
        out_type=jax.ShapeDtypeStruct((NC, N, D), jnp.float32),
        mesh=mesh,
        scratch_types=(
            [pltpu.VMEM((PH, CHUNK), jnp.int32)] * 2
            + [pltpu.VMEM((CHUNK, D), jnp.float32)] * NBUF
            + [pltpu.SemaphoreType.DMA] * (2 * NBUF)
            + [pltpu.VMEM_SHARED((N, D), jnp.float32)]
        ),
    )
    return deg_k, prop_k


BLK = 1000
GRID = 2 * N // BLK


def _tc1_body(x_ref, w_ref, bl_ref, deg_ref, init_ref, hp_ref, dinv_ref):
    y = jnp.dot(x_ref[...], w_ref[...], preferred_element_type=jnp.float32)
    init_ref[...] = y[:, :D] + bl_ref[...][None, :]
    dinv = lax.rsqrt(deg_ref[...])
    dinv_ref[...] = dinv
    hp_ref[...] = y[:, D:] * dinv


_tc1 = pl.pallas_call(
    _tc1_body,
    grid=(GRID,),
    in_specs=[
        pl.BlockSpec((BLK, D), lambda i: (i, 0)),
        pl.BlockSpec((D, 2 * D), lambda i: (0, 0)),
        pl.BlockSpec((D,), lambda i: (0,)),
        pl.BlockSpec((BLK, D), lambda i: (i, 0)),
    ],
    out_specs=[
        pl.BlockSpec((BLK, D), lambda i: (i, 0)),
        pl.BlockSpec((BLK, D), lambda i: (i, 0)),
        pl.BlockSpec((BLK, D), lambda i: (i, 0)),
    ],
    out_shape=[
        jax.ShapeDtypeStruct((2 * N, D), jnp.float32),
        jax.ShapeDtypeStruct((2 * N, D), jnp.float32),
        jax.ShapeDtypeStruct((2 * N, D), jnp.float32),
    ],
)


def _tc2_body(acc_ref, dinv_ref, init_ref, b_ref, wg_ref, hp_ref):
    dinv = dinv_ref[...]
    p = jax.nn.relu(acc_ref[...] * dinv + b_ref[...][None, :])
    z = 0.5 * p + 0.5 * init_ref[...]
    hp_ref[...] = jnp.dot(z, wg_ref[...], preferred_element_type=jnp.float32) * dinv


_tc2 = pl.pallas_call(
    _tc2_body,
    grid=(GRID,),
    in_specs=[
        pl.BlockSpec((BLK, D), lambda i: (i, 0)),
        pl.BlockSpec((BLK, D), lambda i: (i, 0)),
        pl.BlockSpec((BLK, D), lambda i: (i, 0)),
        pl.BlockSpec((D,), lambda i: (0,)),
        pl.BlockSpec((D, D), lambda i: (0, 0)),
    ],
    out_specs=pl.BlockSpec((BLK, D), lambda i: (i, 0)),
    out_shape=jax.ShapeDtypeStruct((2 * N, D), jnp.float32),
)


def _tc3_body(acc_ref, dinv_ref, init_ref, b_ref, out_ref):
    p = jax.nn.relu(acc_ref[...] * dinv_ref[...] + b_ref[...][None, :])
    y = 0.5 * p + 0.5 * init_ref[...]
    nrm = jnp.sqrt(jnp.sum(y * y, axis=1, keepdims=True))
    out_ref[...] = y / jnp.maximum(nrm, 1e-12)


_tc3 = pl.pallas_call(
    _tc3_body,
    grid=(GRID,),
    in_specs=[
        pl.BlockSpec((BLK, D), lambda i: (i, 0)),
        pl.BlockSpec((BLK, D), lambda i: (i, 0)),
        pl.BlockSpec((BLK, D), lambda i: (i, 0)),
        pl.BlockSpec((D,), lambda i: (0,)),
    ],
    out_specs=pl.BlockSpec((BLK, D), lambda i: (i, 0)),
    out_shape=jax.ShapeDtypeStruct((2 * N, D), jnp.float32),
)


def kernel(x1, edge_index1, x2, edge_index2, W_lin, b_lin, W_in, b_in, W_g, b_g):
    srcs = jnp.stack([edge_index1[0], edge_index2[0]]).reshape(
        NC, NS, PHASES, PH, CHUNK)
    dsts = jnp.stack([edge_index1[1], edge_index2[1]]).reshape(
        NC, NS, PHASES, PH, CHUNK)
    dsts_deg = jnp.stack([edge_index1[1], edge_index2[1]]).reshape(
        NC, NS, PHASES_D, PH_D, CH_D)
    ones1 = jnp.ones((640,), jnp.float32)
    ones2 = jnp.ones((128,), jnp.float32)
    _deg_kernel, _prop_kernel = _sc_kernels()

    deg = _deg_kernel(dsts_deg, ones1, ones2)[:, :N]
    deg_bc = jnp.broadcast_to(deg.reshape(2 * N, 1), (2 * N, D))

    x_all = jnp.concatenate([x1, x2], axis=0)
    w_cat = jnp.concatenate([W_lin, W_in], axis=1)
    init, hp0, dinv = _tc1(x_all, w_cat, b_lin, deg_bc)

    def layer(i, carry):
        hp, _ = carry
        acc = _prop_kernel(hp.reshape(NC, N, D), srcs, dsts).reshape(2 * N, D)
        b = jnp.where(i == 0, b_in, b_g)
        hp_next = _tc2(acc, dinv, init, b, W_g)
        return hp_next, acc

    _, acc3 = lax.fori_loop(0, 3, layer, (hp0, jnp.zeros_like(hp0)))
    out = _tc3(acc3, dinv, init, b_g)

    return out[:N], out[N:]
